# baseline (speedup 1.0000x reference)
import jax
import jax.numpy as jnp
from jax import lax
from jax.experimental import pallas as pl
from jax.experimental.pallas import tpu as pltpu
from jax.experimental.pallas import tpu_sc as plsc

N = 10000
D = 128
E = 320000
NC = 2
NS = 16
K = 80
CPE = E // (NS * K)
CPEP = 256
NBUF = 2
HALF = N // NC
ACC = 5120
RPT = ACC // NS
TRASH = 5056
HR = 80


def _sc_agg_body(h_hbm, src_hbm, dst_hbm, parts_hbm, hist_hbm,
                 src_v, dst_v, rows_v, iota_v, agg_s, hsum_s, g0, g1):
    gsems = (g0, g1)
    c = lax.axis_index("c")
    s = lax.axis_index("s")
    lo = c * HALF

    def zrow(r, carry):
        for jj in range(D // 16):
            rows_v[0, r, pl.ds(jj * 16, 16)] = jnp.zeros((16,), jnp.float32)
        return carry
    lax.fori_loop(0, K, zrow, 0)
    zsrc = rows_v.at[0]
    for z in range(RPT // K):
        pltpu.sync_copy(zsrc, agg_s.at[pl.ds(s * RPT + z * K, K)])

    @pl.when((c == 0) & (s == 0))
    def _zero_hsum():
        pltpu.sync_copy(zsrc, hsum_s)

    @pl.when(c == 0)
    def _iota():
        i16v = lax.iota(jnp.int32, 16)

        def istep(i, carry):
            iota_v[pl.ds(i * 16, 16)] = i16v + i * 16
            return carry
        lax.fori_loop(0, HR // 16, istep, 0)

    pltpu.sync_copy(src_hbm.at[s], src_v)
    pltpu.sync_copy(dst_hbm.at[s], dst_v)

    @pl.when(c == 0)
    def _hist():
        def zh(i, carry):
            r = i // (D // 16)
            cc = (i % (D // 16)) * 16
            rows_v[1, r, pl.ds(cc, 16)] = jnp.zeros((16,), jnp.float32)
            return carry
        lax.fori_loop(0, HR * (D // 16), zh, 0)
        ones = jnp.ones((16,), jnp.float32)

        def hstep(t, carry):
            r = t // (K // 16)
            cc = (t % (K // 16)) * 16
            v = dst_v[r, pl.ds(cc, 16)]
            plsc.addupdate_scatter(
                rows_v.at[1],
                [jnp.right_shift(v, 7), jnp.bitwise_and(v, 127)], ones)
            return carry
        lax.fori_loop(0, CPE * (K // 16), hstep, 0)

    def rstep(t, carry):
        r = t // (K // 16)
        cc = (t % (K // 16)) * 16
        v = dst_v[r, pl.ds(cc, 16)]
        inr = (v >= lo) & (v < lo + HALF)
        dst_v[r, pl.ds(cc, 16)] = jnp.where(inr, v - lo, TRASH + (v & 63))
        return carry
    lax.fori_loop(0, CPEP * (K // 16), rstep, 0)

    plsc.subcore_barrier()

    @pl.when(c == 0)
    def _hadd():
        pltpu.sync_copy(rows_v.at[1], hsum_s.at[iota_v], add=True)

    pltpu.async_copy(h_hbm.at[src_v.at[0]], rows_v.at[0], gsems[0])

    def group(g, carry):
        for b in range(NBUF):
            j = g * NBUF + b
            nb = 1 - b
            pltpu.make_async_copy(h_hbm.at[pl.ds(0, K)], rows_v.at[b],
                                  gsems[b]).wait()

            @pl.when(j + 1 < CPEP)
            def _issue_next():
                pltpu.async_copy(h_hbm.at[src_v.at[j + 1]], rows_v.at[nb],
                                 gsems[nb])
            pltpu.sync_copy(rows_v.at[b], agg_s.at[dst_v.at[j]], add=True)
        return carry
    lax.fori_loop(0, CPEP // NBUF, group, 0)

    plsc.subcore_barrier()
    pltpu.sync_copy(agg_s.at[pl.ds(s * RPT, RPT)],
                    parts_hbm.at[c, pl.ds(s * RPT, RPT)])

    @pl.when((c == 0) & (s == 0))
    def _hist_out():
        pltpu.sync_copy(hsum_s, hist_hbm)


_SC_MESH = plsc.VectorSubcoreMesh(core_axis_name="c", subcore_axis_name="s")

_sc_agg = pl.kernel(
    _sc_agg_body,
    out_type=(jax.ShapeDtypeStruct((NC, ACC, D), jnp.float32),
              jax.ShapeDtypeStruct((HR, D), jnp.float32)),
    mesh=_SC_MESH,
    scratch_types=[
        pltpu.VMEM((CPEP, K), jnp.int32),
        pltpu.VMEM((CPEP, K), jnp.int32),
        pltpu.VMEM((NBUF, K, D), jnp.float32),
        pltpu.VMEM((HR,), jnp.int32),
        pltpu.VMEM_SHARED((ACC, D), jnp.float32),
        pltpu.VMEM_SHARED((HR, D), jnp.float32),
        pltpu.SemaphoreType.DMA,
        pltpu.SemaphoreType.DMA,
    ],
    compiler_params=pltpu.CompilerParams(needs_layout_passes=False),
)


RB = 200
NB_HALF = HALF // RB


def _dense_body(parts_ref, deg_ref, h_ref, wl_ref, wr_ref, b_ref, fl_ref,
                out_ref):
    degc = jnp.maximum(deg_ref[...], 1.0)
    agg = parts_ref[0] / degc
    y = (jnp.dot(agg, wl_ref[...], preferred_element_type=jnp.float32)
         + jnp.dot(h_ref[...], wr_ref[...], preferred_element_type=jnp.float32)
         + b_ref[...])
    sp = jnp.maximum(y, 0.0) + jnp.log1p(jnp.exp(-jnp.abs(y)))
    m = y * jnp.tanh(sp)
    out_ref[...] = jnp.where(fl_ref[0, 0] > 0.0, m, y)


_dense = pl.pallas_call(
    _dense_body,
    grid=(N // RB,),
    in_specs=[
        pl.BlockSpec((1, RB, D), lambda i: (i // NB_HALF, i % NB_HALF, 0)),
        pl.BlockSpec((RB, 1), lambda i: (i, 0)),
        pl.BlockSpec((RB, D), lambda i: (i, 0)),
        pl.BlockSpec((D, D), lambda i: (0, 0)),
        pl.BlockSpec((D, D), lambda i: (0, 0)),
        pl.BlockSpec((1, D), lambda i: (0, 0)),
        pl.BlockSpec((1, 1), lambda i: (0, 0)),
    ],
    out_specs=pl.BlockSpec((RB, D), lambda i: (i, 0)),
    out_shape=jax.ShapeDtypeStruct((N, D), jnp.float32),
)


def _fold_bn(Wl, bl, Wr, g, b):
    sc = g * (1.0 / jnp.sqrt(1.0 + 1e-5))
    wlT = (Wl * sc[:, None]).T
    wrT = (Wr * sc[:, None]).T
    bb = (bl * sc + b).reshape(1, D)
    return wlT, wrT, bb


def kernel(x, edge_index, Wl0, bl0, Wr0, g0, b0, Wl1, bl1, Wr1, g1, b1,
           Wl2, bl2, Wr2, g2, b2):
    pad_s = jnp.zeros((NS, CPEP - CPE, K), jnp.int32)
    pad_d = jnp.full((NS, CPEP - CPE, K), -1, jnp.int32)
    src2 = jnp.concatenate([edge_index[0].reshape(NS, CPE, K), pad_s], axis=1)
    dst2 = jnp.concatenate([edge_index[1].reshape(NS, CPE, K), pad_d], axis=1)

    wl0, wr0, bb0 = _fold_bn(Wl0, bl0, Wr0, g0, b0)
    wl1, wr1, bb1 = _fold_bn(Wl1, bl1, Wr1, g1, b1)
    wl2, wr2, bb2 = _fold_bn(Wl2, bl2, Wr2, g2, b2)
    wls = jnp.stack([wl0, wl1, wl2])
    wrs = jnp.stack([wr0, wr1, wr2])
    bbs = jnp.stack([bb0, bb1, bb2])
    fls = jnp.array([1.0, 1.0, 0.0], jnp.float32).reshape(3, 1, 1)

    def step(h, xs):
        wl, wr, bb, fl = xs
        parts, hist = _sc_agg(h, src2, dst2)
        deg3 = hist.reshape(HR * D, 1)
        h2 = _dense(parts, deg3, h, wl, wr, bb, fl)
        return h2, None

    h3, _ = lax.scan(step, x, (wls, wrs, bbs, fls))
    return h3


# speedup vs baseline: 1.0094x; 1.0094x over previous
import jax
import jax.numpy as jnp
from jax import lax
from jax.experimental import pallas as pl
from jax.experimental.pallas import tpu as pltpu
from jax.experimental.pallas import tpu_sc as plsc

N = 10000
D = 128
E = 320000
NC = 2
NS = 16
K = 80
CPE = E // (NS * K)
CPEP = 256
TOTC = 128
NBUF = 4
HALF = N // NC
ACC = 5120
RPT = ACC // NS
TRASH = 5056
HR = 80


def _sc_agg_body(h_hbm, src_hbm, dst_hbm, parts_hbm, hist_hbm,
                 src_v, dst_v, rows_v, iota_v, agg_s, hsum_s,
                 g0, g1, g2, g3):
    gsems = (g0, g1, g2, g3)
    c = lax.axis_index("c")
    s = lax.axis_index("s")
    lo = c * HALF

    def zrow(r, carry):
        for jj in range(D // 16):
            rows_v[0, r, pl.ds(jj * 16, 16)] = jnp.zeros((16,), jnp.float32)
        return carry
    lax.fori_loop(0, K, zrow, 0)
    zsrc = rows_v.at[0]
    for z in range(RPT // K):
        pltpu.sync_copy(zsrc, agg_s.at[pl.ds(s * RPT + z * K, K)])

    @pl.when((c == 0) & (s == 0))
    def _zero_hsum():
        pltpu.sync_copy(zsrc, hsum_s)

    @pl.when(c == 0)
    def _iota():
        i16v = lax.iota(jnp.int32, 16)

        def istep(i, carry):
            iota_v[pl.ds(i * 16, 16)] = i16v + i * 16
            return carry
        lax.fori_loop(0, HR // 16, istep, 0)

    for phase in range(2):
        sl = pl.ds(phase * TOTC, TOTC)
        pltpu.sync_copy(src_hbm.at[s, sl], src_v)
        pltpu.sync_copy(dst_hbm.at[s, sl], dst_v)

        hr = TOTC if phase == 0 else CPE - TOTC

        @pl.when(c == 0)
        def _hist():
            def zh(i, carry):
                r = i // (D // 16)
                cc = (i % (D // 16)) * 16
                rows_v[1, r, pl.ds(cc, 16)] = jnp.zeros((16,), jnp.float32)
                return carry
            lax.fori_loop(0, HR * (D // 16), zh, 0)
            ones = jnp.ones((16,), jnp.float32)

            def hstep(t, carry):
                r = t // (K // 16)
                cc = (t % (K // 16)) * 16
                v = dst_v[r, pl.ds(cc, 16)]
                plsc.addupdate_scatter(
                    rows_v.at[1],
                    [jnp.right_shift(v, 7), jnp.bitwise_and(v, 127)], ones)
                return carry
            lax.fori_loop(0, hr * (K // 16), hstep, 0)

        def rstep(t, carry):
            r = t // (K // 16)
            cc = (t % (K // 16)) * 16
            v = dst_v[r, pl.ds(cc, 16)]
            inr = (v >= lo) & (v < lo + HALF)
            dst_v[r, pl.ds(cc, 16)] = jnp.where(
                inr, v - lo, TRASH + (v & 63))
            return carry
        lax.fori_loop(0, TOTC * (K // 16), rstep, 0)

        if phase == 0:
            plsc.subcore_barrier()

        @pl.when(c == 0)
        def _hadd():
            pltpu.sync_copy(rows_v.at[1], hsum_s.at[iota_v], add=True)

        def group(g, carry):
            descs = []
            for b in range(NBUF):
                j = g * NBUF + b
                descs.append(pltpu.async_copy(h_hbm.at[src_v.at[j]],
                                              rows_v.at[b], gsems[b]))
            for b in range(NBUF):
                j = g * NBUF + b
                descs[b].wait()
                pltpu.sync_copy(rows_v.at[b], agg_s.at[dst_v.at[j]],
                                add=True)
            return carry
        lax.fori_loop(0, TOTC // NBUF, group, 0)

    plsc.subcore_barrier()
    pltpu.sync_copy(agg_s.at[pl.ds(s * RPT, RPT)],
                    parts_hbm.at[c, pl.ds(s * RPT, RPT)])

    @pl.when((c == 0) & (s == 0))
    def _hist_out():
        pltpu.sync_copy(hsum_s, hist_hbm)


_SC_MESH = plsc.VectorSubcoreMesh(core_axis_name="c", subcore_axis_name="s")

_sc_agg = pl.kernel(
    _sc_agg_body,
    out_type=(jax.ShapeDtypeStruct((NC, ACC, D), jnp.float32),
              jax.ShapeDtypeStruct((HR, D), jnp.float32)),
    mesh=_SC_MESH,
    scratch_types=[
        pltpu.VMEM((TOTC, K), jnp.int32),
        pltpu.VMEM((TOTC, K), jnp.int32),
        pltpu.VMEM((NBUF, K, D), jnp.float32),
        pltpu.VMEM((HR,), jnp.int32),
        pltpu.VMEM_SHARED((ACC, D), jnp.float32),
        pltpu.VMEM_SHARED((HR, D), jnp.float32),
        pltpu.SemaphoreType.DMA,
        pltpu.SemaphoreType.DMA,
        pltpu.SemaphoreType.DMA,
        pltpu.SemaphoreType.DMA,
    ],
    compiler_params=pltpu.CompilerParams(needs_layout_passes=False),
)


RB = 200
NB_HALF = HALF // RB


def _dense_body(parts_ref, deg_ref, h_ref, wl_ref, wr_ref, b_ref, fl_ref,
                out_ref):
    degc = jnp.maximum(deg_ref[...], 1.0)
    agg = parts_ref[0] / degc
    y = (jnp.dot(agg, wl_ref[...], preferred_element_type=jnp.float32)
         + jnp.dot(h_ref[...], wr_ref[...], preferred_element_type=jnp.float32)
         + b_ref[...])
    sp = jnp.maximum(y, 0.0) + jnp.log1p(jnp.exp(-jnp.abs(y)))
    m = y * jnp.tanh(sp)
    out_ref[...] = jnp.where(fl_ref[0, 0] > 0.0, m, y)


_dense = pl.pallas_call(
    _dense_body,
    grid=(N // RB,),
    in_specs=[
        pl.BlockSpec((1, RB, D), lambda i: (i // NB_HALF, i % NB_HALF, 0)),
        pl.BlockSpec((RB, 1), lambda i: (i, 0)),
        pl.BlockSpec((RB, D), lambda i: (i, 0)),
        pl.BlockSpec((D, D), lambda i: (0, 0)),
        pl.BlockSpec((D, D), lambda i: (0, 0)),
        pl.BlockSpec((1, D), lambda i: (0, 0)),
        pl.BlockSpec((1, 1), lambda i: (0, 0)),
    ],
    out_specs=pl.BlockSpec((RB, D), lambda i: (i, 0)),
    out_shape=jax.ShapeDtypeStruct((N, D), jnp.float32),
)


def _fold_bn(Wl, bl, Wr, g, b):
    sc = g * (1.0 / jnp.sqrt(1.0 + 1e-5))
    wlT = (Wl * sc[:, None]).T
    wrT = (Wr * sc[:, None]).T
    bb = (bl * sc + b).reshape(1, D)
    return wlT, wrT, bb


def kernel(x, edge_index, Wl0, bl0, Wr0, g0, b0, Wl1, bl1, Wr1, g1, b1,
           Wl2, bl2, Wr2, g2, b2):
    pad_s = jnp.zeros((NS, CPEP - CPE, K), jnp.int32)
    pad_d = jnp.full((NS, CPEP - CPE, K), -1, jnp.int32)
    src2 = jnp.concatenate([edge_index[0].reshape(NS, CPE, K), pad_s], axis=1)
    dst2 = jnp.concatenate([edge_index[1].reshape(NS, CPE, K), pad_d], axis=1)

    wl0, wr0, bb0 = _fold_bn(Wl0, bl0, Wr0, g0, b0)
    wl1, wr1, bb1 = _fold_bn(Wl1, bl1, Wr1, g1, b1)
    wl2, wr2, bb2 = _fold_bn(Wl2, bl2, Wr2, g2, b2)
    wls = jnp.stack([wl0, wl1, wl2])
    wrs = jnp.stack([wr0, wr1, wr2])
    bbs = jnp.stack([bb0, bb1, bb2])
    fls = jnp.array([1.0, 1.0, 0.0], jnp.float32).reshape(3, 1, 1)

    def step(h, xs):
        wl, wr, bb, fl = xs
        parts, hist = _sc_agg(h, src2, dst2)
        deg3 = hist.reshape(HR * D, 1)
        h2 = _dense(parts, deg3, h, wl, wr, bb, fl)
        return h2, None

    h3, _ = lax.scan(step, x, (wls, wrs, bbs, fls))
    return h3


# speedup vs baseline: 1.8734x; 1.8560x over previous
import jax
import jax.numpy as jnp
from jax import lax
from jax.experimental import pallas as pl
from jax.experimental.pallas import tpu as pltpu
from jax.experimental.pallas import tpu_sc as plsc

N = 10000
D = 128
E = 320000
NC = 2
NS = 16
K = 128
EPT = E // NS
CPE = 157
CPEP = 160
HALF = N // NC
ACC = 5120
RPT = ACC // NS
TRASH = 5056
HR = 80


def _sc_agg_body(h_hbm, src_hbm, dst_hbm, parts_hbm, hist_hbm,
                 src_v, dst_v, rows_v, iota_v, agg_s, hsum_s, sem):
    c = lax.axis_index("c")
    s = lax.axis_index("s")

    for hh in range(2):
        sl = pl.ds(hh * (CPEP // 2), CPEP // 2)
        pltpu.sync_copy(src_hbm.at[s, sl], src_v.at[sl])
        pltpu.sync_copy(dst_hbm.at[s, sl], dst_v.at[sl])

    def zrow(r, carry):
        for jj in range(D // 16):
            rows_v[r, pl.ds(jj * 16, 16)] = jnp.zeros((16,), jnp.float32)
        return carry
    lax.fori_loop(0, K, zrow, 0)
    for z in range(RPT // K):
        pltpu.sync_copy(rows_v, agg_s.at[pl.ds(s * RPT + z * K, K)])
    rem = RPT % K
    if rem:
        pltpu.sync_copy(rows_v.at[pl.ds(0, rem)],
                        agg_s.at[pl.ds(s * RPT + (RPT // K) * K, rem)])

    @pl.when((c == 0) & (s == 0))
    def _zero_hsum():
        pltpu.sync_copy(rows_v.at[pl.ds(0, HR)], hsum_s)

    @pl.when(c == 0)
    def _hist():
        ones = jnp.ones((16,), jnp.float32)

        def hstep(t, carry):
            r = t // (K // 16)
            cc = (t % (K // 16)) * 16
            v = dst_v[r, pl.ds(cc, 16)]
            plsc.addupdate_scatter(
                rows_v.at[pl.ds(0, HR)],
                [jnp.right_shift(v, 7), jnp.bitwise_and(v, 127)], ones)
            return carry
        lax.fori_loop(0, EPT // 16, hstep, 0)
        i16 = lax.iota(jnp.int32, 16)

        def istep(i, carry):
            iota_v[pl.ds(i * 16, 16)] = i16 + i * 16
            return carry
        lax.fori_loop(0, HR // 16, istep, 0)

    lo = c * HALF

    def rstep(t, carry):
        r = t // (K // 16)
        cc = (t % (K // 16)) * 16
        v = dst_v[r, pl.ds(cc, 16)]
        inr = (v >= lo) & (v < lo + HALF)
        v2 = jnp.where(inr, v - lo, TRASH + (v & 63))
        dst_v[r, pl.ds(cc, 16)] = v2
        return carry
    lax.fori_loop(0, CPE * (K // 16), rstep, 0)

    plsc.subcore_barrier()

    @pl.when(c == 0)
    def _hadd():
        pltpu.sync_copy(rows_v.at[pl.ds(0, HR)], hsum_s.at[iota_v],
                        add=True)

    def step(j, carry):
        pltpu.async_copy(h_hbm.at[src_v.at[j]], rows_v, sem).wait()
        pltpu.sync_copy(rows_v, agg_s.at[dst_v.at[j]], add=True)
        return carry
    lax.fori_loop(0, CPE, step, 0)

    plsc.subcore_barrier()
    pltpu.sync_copy(agg_s.at[pl.ds(s * RPT, RPT)],
                    parts_hbm.at[c, pl.ds(s * RPT, RPT)])

    @pl.when((c == 0) & (s == 0))
    def _hist_out():
        pltpu.sync_copy(hsum_s, hist_hbm)


_SC_MESH = plsc.VectorSubcoreMesh(core_axis_name="c", subcore_axis_name="s")

_sc_agg = pl.kernel(
    _sc_agg_body,
    out_type=(jax.ShapeDtypeStruct((NC, ACC, D), jnp.float32),
              jax.ShapeDtypeStruct((HR, D), jnp.float32)),
    mesh=_SC_MESH,
    scratch_types=[
        pltpu.VMEM((CPEP, K), jnp.int32),
        pltpu.VMEM((CPEP, K), jnp.int32),
        pltpu.VMEM((K, D), jnp.float32),
        pltpu.VMEM((HR,), jnp.int32),
        pltpu.VMEM_SHARED((ACC, D), jnp.float32),
        pltpu.VMEM_SHARED((HR, D), jnp.float32),
        pltpu.SemaphoreType.DMA,
    ],
    compiler_params=pltpu.CompilerParams(needs_layout_passes=False),
)


RB = 200
NB_HALF = HALF // RB


def _dense_body(parts_ref, deg_ref, h_ref, wl_ref, wr_ref, b_ref, fl_ref,
                out_ref):
    degc = jnp.maximum(deg_ref[...], 1.0)
    agg = parts_ref[0] / degc
    y = (jnp.dot(agg, wl_ref[...], preferred_element_type=jnp.float32)
         + jnp.dot(h_ref[...], wr_ref[...], preferred_element_type=jnp.float32)
         + b_ref[...])
    sp = jnp.maximum(y, 0.0) + jnp.log1p(jnp.exp(-jnp.abs(y)))
    m = y * jnp.tanh(sp)
    out_ref[...] = jnp.where(fl_ref[0, 0] > 0.0, m, y)


_dense = pl.pallas_call(
    _dense_body,
    grid=(N // RB,),
    in_specs=[
        pl.BlockSpec((1, RB, D), lambda i: (i // NB_HALF, i % NB_HALF, 0)),
        pl.BlockSpec((RB, 1), lambda i: (i, 0)),
        pl.BlockSpec((RB, D), lambda i: (i, 0)),
        pl.BlockSpec((D, D), lambda i: (0, 0)),
        pl.BlockSpec((D, D), lambda i: (0, 0)),
        pl.BlockSpec((1, D), lambda i: (0, 0)),
        pl.BlockSpec((1, 1), lambda i: (0, 0)),
    ],
    out_specs=pl.BlockSpec((RB, D), lambda i: (i, 0)),
    out_shape=jax.ShapeDtypeStruct((N, D), jnp.float32),
)


def _fold_bn(Wl, bl, Wr, g, b):
    sc = g * (1.0 / jnp.sqrt(1.0 + 1e-5))
    wlT = (Wl * sc[:, None]).T
    wrT = (Wr * sc[:, None]).T
    bb = (bl * sc + b).reshape(1, D)
    return wlT, wrT, bb


def kernel(x, edge_index, Wl0, bl0, Wr0, g0, b0, Wl1, bl1, Wr1, g1, b1,
           Wl2, bl2, Wr2, g2, b2):
    npad = CPEP * K - EPT
    pad_s = jnp.zeros((NS, npad), jnp.int32)
    pad_d = jnp.full((NS, npad), -1, jnp.int32)
    src2 = jnp.concatenate([edge_index[0].reshape(NS, EPT), pad_s],
                           axis=1).reshape(NS, CPEP, K)
    dst2 = jnp.concatenate([edge_index[1].reshape(NS, EPT), pad_d],
                           axis=1).reshape(NS, CPEP, K)

    wl0, wr0, bb0 = _fold_bn(Wl0, bl0, Wr0, g0, b0)
    wl1, wr1, bb1 = _fold_bn(Wl1, bl1, Wr1, g1, b1)
    wl2, wr2, bb2 = _fold_bn(Wl2, bl2, Wr2, g2, b2)
    wls = jnp.stack([wl0, wl1, wl2])
    wrs = jnp.stack([wr0, wr1, wr2])
    bbs = jnp.stack([bb0, bb1, bb2])
    fls = jnp.array([1.0, 1.0, 0.0], jnp.float32).reshape(3, 1, 1)

    def step(h, xs):
        wl, wr, bb, fl = xs
        parts, hist = _sc_agg(h, src2, dst2)
        deg3 = hist.reshape(HR * D, 1)
        h2 = _dense(parts, deg3, h, wl, wr, bb, fl)
        return h2, None

    h3, _ = lax.scan(step, x, (wls, wrs, bbs, fls))
    return h3


# speedup vs baseline: 2.8645x; 1.5291x over previous
import jax
import jax.numpy as jnp
from jax import lax
from jax.experimental import pallas as pl
from jax.experimental.pallas import tpu as pltpu
from jax.experimental.pallas import tpu_sc as plsc

N = 10000
D = 128
E = 320000
NC = 2
NS = 16
K = 128
EPT = E // NS
CAP = 20480
HALF = N // NC
ACC = 5120
RPT = ACC // NS
TRASH = 5056
HR = 80


def _sc_agg_body(h_hbm, src_hbm, dst_hbm, parts_hbm, hist_hbm,
                 src_v, dst_v, rows_v, iota_v, agg_s, hsum_s, sem):
    c = lax.axis_index("c")
    s = lax.axis_index("s")

    for hh in range(2):
        pltpu.sync_copy(src_hbm.at[pl.ds(s * CAP + hh * (CAP // 2),
                                         CAP // 2)],
                        src_v.at[pl.ds(hh * (CAP // 2), CAP // 2)])
        pltpu.sync_copy(dst_hbm.at[pl.ds(s * CAP + hh * (CAP // 2),
                                         CAP // 2)],
                        dst_v.at[pl.ds(hh * (CAP // 2), CAP // 2)])

    def zrow(r, carry):
        for jj in range(D // 16):
            rows_v[r, pl.ds(jj * 16, 16)] = jnp.zeros((16,), jnp.float32)
        return carry
    lax.fori_loop(0, K, zrow, 0)
    for z in range(RPT // K):
        pltpu.sync_copy(rows_v, agg_s.at[pl.ds(s * RPT + z * K, K)])
    rem = RPT % K
    if rem:
        pltpu.sync_copy(rows_v.at[pl.ds(0, rem)],
                        agg_s.at[pl.ds(s * RPT + (RPT // K) * K, rem)])

    @pl.when((c == 0) & (s == 0))
    def _zero_hsum():
        pltpu.sync_copy(rows_v.at[pl.ds(0, HR)], hsum_s)

    @pl.when(c == 0)
    def _hist():
        ones = jnp.ones((16,), jnp.float32)

        def hstep(t, carry):
            v = dst_v[pl.ds(t * 16, 16)]
            plsc.addupdate_scatter(
                rows_v.at[pl.ds(0, HR)],
                [jnp.right_shift(v, 7), jnp.bitwise_and(v, 127)], ones)
            return carry
        lax.fori_loop(0, EPT // 16, hstep, 0)
        i16 = lax.iota(jnp.int32, 16)

        def istep(i, carry):
            iota_v[pl.ds(i * 16, 16)] = i16 + i * 16
            return carry
        lax.fori_loop(0, HR // 16, istep, 0)

    lo = c * HALF

    def cstep(t, off):
        vd = dst_v[pl.ds(t * 16, 16)]
        vs = src_v[pl.ds(t * 16, 16)]
        m = (vd >= lo) & (vd < lo + HALF)
        plsc.store_compressed(dst_v.at[pl.ds(off, 16)], vd - lo, mask=m)
        plsc.store_compressed(src_v.at[pl.ds(off, 16)], vs, mask=m)
        cnt = plsc.all_reduce_population_count(m)
        return off + cnt[0]
    off = lax.fori_loop(0, EPT // 16, cstep, jnp.int32(0))

    i16 = lax.iota(jnp.int32, 16)
    z16 = jnp.zeros((16,), jnp.int32)
    for kk in range(K // 16):
        dst_v[pl.ds(off + kk * 16, 16)] = TRASH + i16
        src_v[pl.ds(off + kk * 16, 16)] = z16
    nchunks = (off + K - 1) // K

    plsc.subcore_barrier()

    @pl.when(c == 0)
    def _hadd():
        pltpu.sync_copy(rows_v.at[pl.ds(0, HR)], hsum_s.at[iota_v],
                        add=True)

    def step(j, carry):
        pltpu.async_copy(h_hbm.at[src_v.at[pl.ds(j * K, K)]], rows_v,
                         sem).wait()
        pltpu.sync_copy(rows_v, agg_s.at[dst_v.at[pl.ds(j * K, K)]],
                        add=True)
        return carry
    lax.fori_loop(0, nchunks, step, 0)

    plsc.subcore_barrier()
    pltpu.sync_copy(agg_s.at[pl.ds(s * RPT, RPT)],
                    parts_hbm.at[c, pl.ds(s * RPT, RPT)])

    @pl.when((c == 0) & (s == 0))
    def _hist_out():
        pltpu.sync_copy(hsum_s, hist_hbm)


_SC_MESH = plsc.VectorSubcoreMesh(core_axis_name="c", subcore_axis_name="s")

_sc_agg = pl.kernel(
    _sc_agg_body,
    out_type=(jax.ShapeDtypeStruct((NC, ACC, D), jnp.float32),
              jax.ShapeDtypeStruct((HR, D), jnp.float32)),
    mesh=_SC_MESH,
    scratch_types=[
        pltpu.VMEM((CAP,), jnp.int32),
        pltpu.VMEM((CAP,), jnp.int32),
        pltpu.VMEM((K, D), jnp.float32),
        pltpu.VMEM((HR,), jnp.int32),
        pltpu.VMEM_SHARED((ACC, D), jnp.float32),
        pltpu.VMEM_SHARED((HR, D), jnp.float32),
        pltpu.SemaphoreType.DMA,
    ],
    compiler_params=pltpu.CompilerParams(needs_layout_passes=False),
)


RB = 200
NB_HALF = HALF // RB


def _dense_body(parts_ref, deg_ref, h_ref, wl_ref, wr_ref, b_ref, fl_ref,
                out_ref):
    degc = jnp.maximum(deg_ref[...], 1.0)
    agg = parts_ref[0] / degc
    y = (jnp.dot(agg, wl_ref[...], preferred_element_type=jnp.float32)
         + jnp.dot(h_ref[...], wr_ref[...], preferred_element_type=jnp.float32)
         + b_ref[...])
    sp = jnp.maximum(y, 0.0) + jnp.log1p(jnp.exp(-jnp.abs(y)))
    m = y * jnp.tanh(sp)
    out_ref[...] = jnp.where(fl_ref[0, 0] > 0.0, m, y)


_dense = pl.pallas_call(
    _dense_body,
    grid=(N // RB,),
    in_specs=[
        pl.BlockSpec((1, RB, D), lambda i: (i // NB_HALF, i % NB_HALF, 0)),
        pl.BlockSpec((RB, 1), lambda i: (i, 0)),
        pl.BlockSpec((RB, D), lambda i: (i, 0)),
        pl.BlockSpec((D, D), lambda i: (0, 0)),
        pl.BlockSpec((D, D), lambda i: (0, 0)),
        pl.BlockSpec((1, D), lambda i: (0, 0)),
        pl.BlockSpec((1, 1), lambda i: (0, 0)),
    ],
    out_specs=pl.BlockSpec((RB, D), lambda i: (i, 0)),
    out_shape=jax.ShapeDtypeStruct((N, D), jnp.float32),
)


def _fold_bn(Wl, bl, Wr, g, b):
    sc = g * (1.0 / jnp.sqrt(1.0 + 1e-5))
    wlT = (Wl * sc[:, None]).T
    wrT = (Wr * sc[:, None]).T
    bb = (bl * sc + b).reshape(1, D)
    return wlT, wrT, bb


def kernel(x, edge_index, Wl0, bl0, Wr0, g0, b0, Wl1, bl1, Wr1, g1, b1,
           Wl2, bl2, Wr2, g2, b2):
    pad = jnp.zeros((NS, CAP - EPT), jnp.int32)
    src2 = jnp.concatenate([edge_index[0].reshape(NS, EPT), pad],
                           axis=1).reshape(NS * CAP)
    dst2 = jnp.concatenate([edge_index[1].reshape(NS, EPT), pad],
                           axis=1).reshape(NS * CAP)

    wl0, wr0, bb0 = _fold_bn(Wl0, bl0, Wr0, g0, b0)
    wl1, wr1, bb1 = _fold_bn(Wl1, bl1, Wr1, g1, b1)
    wl2, wr2, bb2 = _fold_bn(Wl2, bl2, Wr2, g2, b2)
    wls = jnp.stack([wl0, wl1, wl2])
    wrs = jnp.stack([wr0, wr1, wr2])
    bbs = jnp.stack([bb0, bb1, bb2])
    fls = jnp.array([1.0, 1.0, 0.0], jnp.float32).reshape(3, 1, 1)

    def step(h, xs):
        wl, wr, bb, fl = xs
        parts, hist = _sc_agg(h, src2, dst2)
        deg3 = hist.reshape(HR * D, 1)
        h2 = _dense(parts, deg3, h, wl, wr, bb, fl)
        return h2, None

    h3, _ = lax.scan(step, x, (wls, wrs, bbs, fls))
    return h3


# speedup vs baseline: 2.9366x; 1.0252x over previous
import jax
import jax.numpy as jnp
from jax import lax
from jax.experimental import pallas as pl
from jax.experimental.pallas import tpu as pltpu
from jax.experimental.pallas import tpu_sc as plsc

N = 10000
D = 128
E = 320000
NC = 2
NS = 16
K = 80
EPT = E // NS
CAP = 20480
HALF = N // NC
ACC = 5120
RPT = ACC // NS
TRASH = 5056
HR = 80


def _sc_agg_body(h_hbm, src_hbm, dst_hbm, parts_hbm, hist_hbm,
                 src_v, dst_v, rows_v, iota_v, agg_s, hsum_s, sem):
    c = lax.axis_index("c")
    s = lax.axis_index("s")

    for hh in range(2):
        pltpu.sync_copy(src_hbm.at[pl.ds(s * CAP + hh * (CAP // 2),
                                         CAP // 2)],
                        src_v.at[pl.ds(hh * (CAP // 2), CAP // 2)])
        pltpu.sync_copy(dst_hbm.at[pl.ds(s * CAP + hh * (CAP // 2),
                                         CAP // 2)],
                        dst_v.at[pl.ds(hh * (CAP // 2), CAP // 2)])

    def zrow(r, carry):
        for jj in range(D // 16):
            rows_v[r, pl.ds(jj * 16, 16)] = jnp.zeros((16,), jnp.float32)
        return carry
    lax.fori_loop(0, K, zrow, 0)
    for z in range(RPT // K):
        pltpu.sync_copy(rows_v, agg_s.at[pl.ds(s * RPT + z * K, K)])
    rem = RPT % K
    if rem:
        pltpu.sync_copy(rows_v.at[pl.ds(0, rem)],
                        agg_s.at[pl.ds(s * RPT + (RPT // K) * K, rem)])

    @pl.when((c == 0) & (s == 0))
    def _zero_hsum():
        pltpu.sync_copy(rows_v.at[pl.ds(0, HR)], hsum_s)

    @pl.when(c == 0)
    def _hist():
        ones = jnp.ones((16,), jnp.float32)

        def hstep(t, carry):
            v = dst_v[pl.ds(t * 16, 16)]
            plsc.addupdate_scatter(
                rows_v.at[pl.ds(0, HR)],
                [jnp.right_shift(v, 7), jnp.bitwise_and(v, 127)], ones)
            return carry
        lax.fori_loop(0, EPT // 16, hstep, 0)
        i16 = lax.iota(jnp.int32, 16)

        def istep(i, carry):
            iota_v[pl.ds(i * 16, 16)] = i16 + i * 16
            return carry
        lax.fori_loop(0, HR // 16, istep, 0)

    lo = c * HALF

    def cstep(t, off):
        vd = dst_v[pl.ds(t * 16, 16)]
        vs = src_v[pl.ds(t * 16, 16)]
        m = (vd >= lo) & (vd < lo + HALF)
        plsc.store_compressed(dst_v.at[pl.ds(off, 16)], vd - lo, mask=m)
        plsc.store_compressed(src_v.at[pl.ds(off, 16)], vs, mask=m)
        cnt = plsc.all_reduce_population_count(m)
        return off + cnt[0]
    off = lax.fori_loop(0, EPT // 16, cstep, jnp.int32(0))

    i16 = lax.iota(jnp.int32, 16)
    z16 = jnp.zeros((16,), jnp.int32)
    for kk in range(K // 16):
        dst_v[pl.ds(off + kk * 16, 16)] = TRASH + i16
        src_v[pl.ds(off + kk * 16, 16)] = z16
    nchunks = (off + K - 1) // K

    plsc.subcore_barrier()

    @pl.when(c == 0)
    def _hadd():
        pltpu.sync_copy(rows_v.at[pl.ds(0, HR)], hsum_s.at[iota_v],
                        add=True)

    def step(j, carry):
        pltpu.async_copy(h_hbm.at[src_v.at[pl.ds(j * K, K)]], rows_v,
                         sem).wait()
        pltpu.sync_copy(rows_v, agg_s.at[dst_v.at[pl.ds(j * K, K)]],
                        add=True)
        return carry
    lax.fori_loop(0, nchunks, step, 0)

    plsc.subcore_barrier()
    pltpu.sync_copy(agg_s.at[pl.ds(s * RPT, RPT)],
                    parts_hbm.at[c, pl.ds(s * RPT, RPT)])

    @pl.when((c == 0) & (s == 0))
    def _hist_out():
        pltpu.sync_copy(hsum_s, hist_hbm)


_SC_MESH = plsc.VectorSubcoreMesh(core_axis_name="c", subcore_axis_name="s")

_sc_agg = pl.kernel(
    _sc_agg_body,
    out_type=(jax.ShapeDtypeStruct((NC, ACC, D), jnp.float32),
              jax.ShapeDtypeStruct((HR, D), jnp.float32)),
    mesh=_SC_MESH,
    scratch_types=[
        pltpu.VMEM((CAP,), jnp.int32),
        pltpu.VMEM((CAP,), jnp.int32),
        pltpu.VMEM((K, D), jnp.float32),
        pltpu.VMEM((HR,), jnp.int32),
        pltpu.VMEM_SHARED((ACC, D), jnp.float32),
        pltpu.VMEM_SHARED((HR, D), jnp.float32),
        pltpu.SemaphoreType.DMA,
    ],
    compiler_params=pltpu.CompilerParams(needs_layout_passes=False),
)


RB = 200
NB_HALF = HALF // RB


def _dense_body(parts_ref, deg_ref, h_ref, wl_ref, wr_ref, b_ref, fl_ref,
                out_ref):
    degc = jnp.maximum(deg_ref[...], 1.0)
    agg = parts_ref[0] / degc
    y = (jnp.dot(agg, wl_ref[...], preferred_element_type=jnp.float32)
         + jnp.dot(h_ref[...], wr_ref[...], preferred_element_type=jnp.float32)
         + b_ref[...])
    sp = jnp.maximum(y, 0.0) + jnp.log1p(jnp.exp(-jnp.abs(y)))
    m = y * jnp.tanh(sp)
    out_ref[...] = jnp.where(fl_ref[0, 0] > 0.0, m, y)


_dense = pl.pallas_call(
    _dense_body,
    grid=(N // RB,),
    in_specs=[
        pl.BlockSpec((1, RB, D), lambda i: (i // NB_HALF, i % NB_HALF, 0)),
        pl.BlockSpec((RB, 1), lambda i: (i, 0)),
        pl.BlockSpec((RB, D), lambda i: (i, 0)),
        pl.BlockSpec((D, D), lambda i: (0, 0)),
        pl.BlockSpec((D, D), lambda i: (0, 0)),
        pl.BlockSpec((1, D), lambda i: (0, 0)),
        pl.BlockSpec((1, 1), lambda i: (0, 0)),
    ],
    out_specs=pl.BlockSpec((RB, D), lambda i: (i, 0)),
    out_shape=jax.ShapeDtypeStruct((N, D), jnp.float32),
)


def _fold_bn(Wl, bl, Wr, g, b):
    sc = g * (1.0 / jnp.sqrt(1.0 + 1e-5))
    wlT = (Wl * sc[:, None]).T
    wrT = (Wr * sc[:, None]).T
    bb = (bl * sc + b).reshape(1, D)
    return wlT, wrT, bb


def kernel(x, edge_index, Wl0, bl0, Wr0, g0, b0, Wl1, bl1, Wr1, g1, b1,
           Wl2, bl2, Wr2, g2, b2):
    pad = jnp.zeros((NS, CAP - EPT), jnp.int32)
    src2 = jnp.concatenate([edge_index[0].reshape(NS, EPT), pad],
                           axis=1).reshape(NS * CAP)
    dst2 = jnp.concatenate([edge_index[1].reshape(NS, EPT), pad],
                           axis=1).reshape(NS * CAP)

    wl0, wr0, bb0 = _fold_bn(Wl0, bl0, Wr0, g0, b0)
    wl1, wr1, bb1 = _fold_bn(Wl1, bl1, Wr1, g1, b1)
    wl2, wr2, bb2 = _fold_bn(Wl2, bl2, Wr2, g2, b2)
    wls = jnp.stack([wl0, wl1, wl2])
    wrs = jnp.stack([wr0, wr1, wr2])
    bbs = jnp.stack([bb0, bb1, bb2])
    fls = jnp.array([1.0, 1.0, 0.0], jnp.float32).reshape(3, 1, 1)

    def step(h, xs):
        wl, wr, bb, fl = xs
        parts, hist = _sc_agg(h, src2, dst2)
        deg3 = hist.reshape(HR * D, 1)
        h2 = _dense(parts, deg3, h, wl, wr, bb, fl)
        return h2, None

    h3, _ = lax.scan(step, x, (wls, wrs, bbs, fls))
    return h3


# speedup vs baseline: 3.1560x; 1.0747x over previous
import jax
import jax.numpy as jnp
from jax import lax
from jax.experimental import pallas as pl
from jax.experimental.pallas import tpu as pltpu
from jax.experimental.pallas import tpu_sc as plsc

N = 10000
D = 128
E = 320000
NC = 2
NS = 16
K = 80
EPT = E // NS
CAP = 20480
HALF = N // NC
ACC = 5120
RPT = ACC // NS
TRASH = 5056
HR = 80


def _sc_agg_body(h_hbm, src_hbm, dst_hbm, parts_hbm, hist_hbm,
                 src_v, dst_v, rows_v, iota_v, agg_s, hsum_s, sem):
    c = lax.axis_index("c")
    s = lax.axis_index("s")

    for hh in range(2):
        pltpu.sync_copy(src_hbm.at[pl.ds(s * CAP + hh * (CAP // 2),
                                         CAP // 2)],
                        src_v.at[pl.ds(hh * (CAP // 2), CAP // 2)])
        pltpu.sync_copy(dst_hbm.at[pl.ds(s * CAP + hh * (CAP // 2),
                                         CAP // 2)],
                        dst_v.at[pl.ds(hh * (CAP // 2), CAP // 2)])

    def zrow(r, carry):
        for jj in range(D // 16):
            rows_v[r, pl.ds(jj * 16, 16)] = jnp.zeros((16,), jnp.float32)
        return carry
    lax.fori_loop(0, K, zrow, 0)
    for z in range(RPT // K):
        pltpu.sync_copy(rows_v, agg_s.at[pl.ds(s * RPT + z * K, K)])
    rem = RPT % K
    if rem:
        pltpu.sync_copy(rows_v.at[pl.ds(0, rem)],
                        agg_s.at[pl.ds(s * RPT + (RPT // K) * K, rem)])

    @pl.when((c == 0) & (s == 0))
    def _zero_hsum():
        pltpu.sync_copy(rows_v.at[pl.ds(0, HR)], hsum_s)

    @pl.when(c == 0)
    def _hist():
        ones = jnp.ones((16,), jnp.float32)

        def hstep(t, carry):
            v = dst_v[pl.ds(t * 16, 16)]
            plsc.addupdate_scatter(
                rows_v.at[pl.ds(0, HR)],
                [jnp.right_shift(v, 7), jnp.bitwise_and(v, 127)], ones)
            return carry
        lax.fori_loop(0, EPT // 16, hstep, 0)
        i16 = lax.iota(jnp.int32, 16)

        def istep(i, carry):
            iota_v[pl.ds(i * 16, 16)] = i16 + i * 16
            return carry
        lax.fori_loop(0, HR // 16, istep, 0)

    lo = c * HALF

    def cstep(t, off):
        vd = dst_v[pl.ds(t * 16, 16)]
        vs = src_v[pl.ds(t * 16, 16)]
        m = (vd >= lo) & (vd < lo + HALF)
        plsc.store_compressed(dst_v.at[pl.ds(off, 16)], vd - lo, mask=m)
        plsc.store_compressed(src_v.at[pl.ds(off, 16)], vs, mask=m)
        cnt = plsc.all_reduce_population_count(m)
        return off + cnt[0]
    off = lax.fori_loop(0, EPT // 16, cstep, jnp.int32(0))

    i16 = lax.iota(jnp.int32, 16)
    z16 = jnp.zeros((16,), jnp.int32)
    for kk in range(K // 16):
        dst_v[pl.ds(off + kk * 16, 16)] = TRASH + i16
        src_v[pl.ds(off + kk * 16, 16)] = z16
    nchunks = (off + K - 1) // K

    plsc.subcore_barrier()

    @pl.when(c == 0)
    def _hadd():
        pltpu.sync_copy(rows_v.at[pl.ds(0, HR)], hsum_s.at[iota_v],
                        add=True)

    def step(j, carry):
        pltpu.async_copy(h_hbm.at[src_v.at[pl.ds(j * K, K)]], rows_v,
                         sem).wait()
        pltpu.sync_copy(rows_v, agg_s.at[dst_v.at[pl.ds(j * K, K)]],
                        add=True)
        return carry
    lax.fori_loop(0, nchunks, step, 0)

    plsc.subcore_barrier()
    pltpu.sync_copy(agg_s.at[pl.ds(s * RPT, RPT)],
                    parts_hbm.at[c, pl.ds(s * RPT, RPT)])

    @pl.when((c == 0) & (s == 0))
    def _hist_out():
        pltpu.sync_copy(hsum_s, hist_hbm)


_SC_MESH = plsc.VectorSubcoreMesh(core_axis_name="c", subcore_axis_name="s")

_sc_agg = pl.kernel(
    _sc_agg_body,
    out_type=(jax.ShapeDtypeStruct((NC, ACC, D), jnp.float32),
              jax.ShapeDtypeStruct((HR, D), jnp.float32)),
    mesh=_SC_MESH,
    scratch_types=[
        pltpu.VMEM((CAP,), jnp.int32),
        pltpu.VMEM((CAP,), jnp.int32),
        pltpu.VMEM((K, D), jnp.float32),
        pltpu.VMEM((HR,), jnp.int32),
        pltpu.VMEM_SHARED((ACC, D), jnp.float32),
        pltpu.VMEM_SHARED((HR, D), jnp.float32),
        pltpu.SemaphoreType.DMA,
    ],
    compiler_params=pltpu.CompilerParams(needs_layout_passes=False),
)


RB = 1000
NB_HALF = HALF // RB


def _dense_body(parts_ref, deg_ref, h_ref, wl_ref, wr_ref, b_ref, fl_ref,
                out_ref):
    degc = jnp.maximum(deg_ref[...], 1.0)
    agg = parts_ref[0] / degc
    y = (jnp.dot(agg, wl_ref[...], preferred_element_type=jnp.float32)
         + jnp.dot(h_ref[...], wr_ref[...], preferred_element_type=jnp.float32)
         + b_ref[...])
    sp = jnp.maximum(y, 0.0) + jnp.log1p(jnp.exp(-jnp.abs(y)))
    m = y * jnp.tanh(sp)
    out_ref[...] = jnp.where(fl_ref[0, 0] > 0.0, m, y)


_dense = pl.pallas_call(
    _dense_body,
    grid=(N // RB,),
    in_specs=[
        pl.BlockSpec((1, RB, D), lambda i: (i // NB_HALF, i % NB_HALF, 0)),
        pl.BlockSpec((RB, 1), lambda i: (i, 0)),
        pl.BlockSpec((RB, D), lambda i: (i, 0)),
        pl.BlockSpec((D, D), lambda i: (0, 0)),
        pl.BlockSpec((D, D), lambda i: (0, 0)),
        pl.BlockSpec((1, D), lambda i: (0, 0)),
        pl.BlockSpec((1, 1), lambda i: (0, 0)),
    ],
    out_specs=pl.BlockSpec((RB, D), lambda i: (i, 0)),
    out_shape=jax.ShapeDtypeStruct((N, D), jnp.float32),
)


def _fold_bn(Wl, bl, Wr, g, b):
    sc = g * (1.0 / jnp.sqrt(1.0 + 1e-5))
    wlT = (Wl * sc[:, None]).T
    wrT = (Wr * sc[:, None]).T
    bb = (bl * sc + b).reshape(1, D)
    return wlT, wrT, bb


def kernel(x, edge_index, Wl0, bl0, Wr0, g0, b0, Wl1, bl1, Wr1, g1, b1,
           Wl2, bl2, Wr2, g2, b2):
    pad = jnp.zeros((NS, CAP - EPT), jnp.int32)
    src2 = jnp.concatenate([edge_index[0].reshape(NS, EPT), pad],
                           axis=1).reshape(NS * CAP)
    dst2 = jnp.concatenate([edge_index[1].reshape(NS, EPT), pad],
                           axis=1).reshape(NS * CAP)

    wl0, wr0, bb0 = _fold_bn(Wl0, bl0, Wr0, g0, b0)
    wl1, wr1, bb1 = _fold_bn(Wl1, bl1, Wr1, g1, b1)
    wl2, wr2, bb2 = _fold_bn(Wl2, bl2, Wr2, g2, b2)
    wls = jnp.stack([wl0, wl1, wl2])
    wrs = jnp.stack([wr0, wr1, wr2])
    bbs = jnp.stack([bb0, bb1, bb2])
    fls = jnp.array([1.0, 1.0, 0.0], jnp.float32).reshape(3, 1, 1)

    def step(h, xs):
        wl, wr, bb, fl = xs
        parts, hist = _sc_agg(h, src2, dst2)
        deg3 = hist.reshape(HR * D, 1)
        h2 = _dense(parts, deg3, h, wl, wr, bb, fl)
        return h2, None

    h3, _ = lax.scan(step, x, (wls, wrs, bbs, fls))
    return h3
